# baseline (speedup 1.0000x reference)
import functools

import jax
import jax.numpy as jnp
import numpy as np
from jax import lax
from jax.experimental import pallas as pl
from jax.experimental.pallas import tpu as pltpu
from jax.experimental.pallas import tpu_sc as plsc


def _sin_pe(max_len, d):
    pos = np.arange(max_len, dtype=np.float32)[:, None]
    div = np.exp(np.arange(0, d, 2, dtype=np.float32) * (-np.log(10000.0) / d))
    pe = np.zeros((max_len, d), dtype=np.float32)
    pe[:, 0::2] = np.sin(pos * div)
    pe[:, 1::2] = np.cos(pos * div)
    return jnp.asarray(pe)


@functools.lru_cache(maxsize=None)
def _build(B, L, D, V):
    info = plsc.get_sparse_core_info()
    NC, NS = info.num_cores, info.num_subcores
    NW = NC * NS
    N = B * L
    assert N % NW == 0
    b_per_w = N // NW
    assert b_per_w % L == 0
    n_chunks = b_per_w // L
    mesh = plsc.VectorSubcoreMesh(core_axis_name="c", subcore_axis_name="s")

    @functools.partial(
        pl.kernel,
        out_type=jax.ShapeDtypeStruct((N, D), jnp.float32),
        mesh=mesh,
        scratch_types=[
            pltpu.VMEM((b_per_w,), jnp.int32),
            pltpu.VMEM((L, D), jnp.float32),
            pltpu.VMEM((L, D), jnp.float32),
            pltpu.SemaphoreType.DMA,
        ],
        compiler_params=pltpu.CompilerParams(use_tc_tiling_on_sc=False),
    )
    def emb(idx_hbm, pe_hbm, table_hbm, out_hbm, idx_v, pe_v, buf, sem):
        wid = lax.axis_index("s") * NC + lax.axis_index("c")
        base = wid * b_per_w
        pltpu.sync_copy(idx_hbm.at[pl.ds(base, b_per_w)], idx_v)
        pltpu.sync_copy(pe_hbm, pe_v)

        def chunk(c, carry):
            off = pl.multiple_of(c * L, 8)
            g0 = pltpu.async_copy(
                table_hbm.at[idx_v.at[pl.ds(off, 128)]], buf.at[pl.ds(0, 128)], sem
            )
            g1 = pltpu.async_copy(
                table_hbm.at[idx_v.at[pl.ds(off + 128, L - 128)]],
                buf.at[pl.ds(128, L - 128)],
                sem,
            )
            g0.wait()
            g1.wait()

            def addrow(r, carry2):
                for j in range(D // 16):
                    s = pl.ds(j * 16, 16)
                    buf[r, s] = buf[r, s] + pe_v[r, s]
                return carry2

            lax.fori_loop(0, L, addrow, 0)
            pltpu.sync_copy(buf, out_hbm.at[pl.ds(base + off, L)])
            return carry

        lax.fori_loop(0, n_chunks, chunk, 0)

    return emb


def kernel(x, table):
    B, L = x.shape
    V, D = table.shape
    idx = x.reshape(-1).astype(jnp.int32)
    pe = _sin_pe(L, D)
    out = _build(B, L, D, V)(idx, pe, table)
    return out.reshape(B, L, D)


# speedup vs baseline: 1.1481x; 1.1481x over previous
import functools

import jax
import jax.numpy as jnp
import numpy as np
from jax import lax
from jax.experimental import pallas as pl
from jax.experimental.pallas import tpu as pltpu
from jax.experimental.pallas import tpu_sc as plsc

_NBUF = 4
_LOOKAHEAD = 2


def _sin_pe(max_len, d):
    pos = np.arange(max_len, dtype=np.float32)[:, None]
    div = np.exp(np.arange(0, d, 2, dtype=np.float32) * (-np.log(10000.0) / d))
    pe = np.zeros((max_len, d), dtype=np.float32)
    pe[:, 0::2] = np.sin(pos * div)
    pe[:, 1::2] = np.cos(pos * div)
    return jnp.asarray(pe)


@functools.lru_cache(maxsize=None)
def _build(B, L, D, V):
    info = plsc.get_sparse_core_info()
    NC, NS = info.num_cores, info.num_subcores
    NW = NC * NS
    N = B * L
    assert N % NW == 0
    b_per_w = N // NW
    assert b_per_w % L == 0
    n_chunks = b_per_w // L
    assert n_chunks % _NBUF == 0
    G0 = 128
    G1 = L - G0
    mesh = plsc.VectorSubcoreMesh(core_axis_name="c", subcore_axis_name="s")

    @functools.partial(
        pl.kernel,
        out_type=jax.ShapeDtypeStruct((N, D), jnp.float32),
        mesh=mesh,
        scratch_types=[
            pltpu.VMEM((b_per_w,), jnp.int32),
            pltpu.VMEM((L, D), jnp.float32),
        ]
        + [pltpu.VMEM((L, D), jnp.float32) for _ in range(_NBUF)]
        + [pltpu.SemaphoreType.DMA for _ in range(2 * _NBUF)],
        compiler_params=pltpu.CompilerParams(use_tc_tiling_on_sc=False),
    )
    def emb(idx_hbm, pe_hbm, table_hbm, out_hbm, idx_v, pe_v, *bufs_and_sems):
        bufs = bufs_and_sems[:_NBUF]
        sem_g = bufs_and_sems[_NBUF : 2 * _NBUF]
        sem_o = bufs_and_sems[2 * _NBUF :]
        wid = lax.axis_index("s") * NC + lax.axis_index("c")
        base = wid * b_per_w
        pltpu.sync_copy(idx_hbm.at[pl.ds(base, b_per_w)], idx_v)
        pltpu.sync_copy(pe_hbm, pe_v)

        def start_gather(c, b):
            off = pl.multiple_of(c * L, 8)
            pltpu.async_copy(
                table_hbm.at[idx_v.at[pl.ds(off, G0)]], bufs[b].at[pl.ds(0, G0)], sem_g[b]
            )
            pltpu.async_copy(
                table_hbm.at[idx_v.at[pl.ds(off + G0, G1)]],
                bufs[b].at[pl.ds(G0, G1)],
                sem_g[b],
            )

        def wait_gather(b):
            pltpu.make_async_copy(
                table_hbm.at[idx_v.at[pl.ds(0, G0)]], bufs[b].at[pl.ds(0, G0)], sem_g[b]
            ).wait()
            pltpu.make_async_copy(
                table_hbm.at[idx_v.at[pl.ds(0, G1)]], bufs[b].at[pl.ds(G0, G1)], sem_g[b]
            ).wait()

        def wait_out(b):
            pltpu.make_async_copy(bufs[b], out_hbm.at[pl.ds(0, L)], sem_o[b]).wait()

        for b in range(_LOOKAHEAD):
            start_gather(b, b)

        rows_per_it = 4
        assert L % rows_per_it == 0

        def grp_body(grp, carry):
            c_base = grp * _NBUF
            for bb in range(_NBUF):
                c = c_base + bb
                buf = bufs[bb]
                wait_gather(bb)

                def addrows(r4, carry2):
                    r0 = r4 * rows_per_it
                    for dr in range(rows_per_it):
                        r = r0 + dr
                        for j in range(D // 16):
                            s = pl.ds(j * 16, 16)
                            plsc.addupdate(buf.at[r, s], pe_v[r, s])
                    return carry2

                lax.fori_loop(0, L // rows_per_it, addrows, 0)
                off = pl.multiple_of(base + c * L, 8)
                pltpu.async_copy(buf, out_hbm.at[pl.ds(off, L)], sem_o[bb])

                c2 = c + _LOOKAHEAD
                b2 = (bb + _LOOKAHEAD) % _NBUF

                @pl.when(jnp.logical_and(c2 < n_chunks, c2 >= _NBUF))
                def _():
                    wait_out(b2)

                @pl.when(c2 < n_chunks)
                def _():
                    start_gather(c2, b2)

            return carry

        lax.fori_loop(0, n_chunks // _NBUF, grp_body, 0)
        for b in range(_NBUF):
            wait_out(b)

    return emb


def kernel(x, table):
    B, L = x.shape
    V, D = table.shape
    idx = x.reshape(-1).astype(jnp.int32)
    pe = _sin_pe(L, D)
    out = _build(B, L, D, V)(idx, pe, table)
    return out.reshape(B, L, D)
